# baseline (speedup 1.0000x reference)
import functools
import math

import jax
import jax.numpy as jnp
from jax import lax
from jax.experimental import pallas as pl
from jax.experimental.pallas import tpu as pltpu
from jax.experimental.pallas import tpu_sc as plsc

MODEL_DIM = 128
SCALE = math.sqrt(float(MODEL_DIM))


def _scale_body(lut_ref, out_ref):
    out_ref[...] = lut_ref[...] * SCALE


@functools.lru_cache(maxsize=None)
def _make_scale(V, D):
    rows = 1000
    assert V % rows == 0
    return pl.pallas_call(
        _scale_body,
        out_shape=jax.ShapeDtypeStruct((V, D), jnp.float32),
        grid=(V // rows,),
        in_specs=[pl.BlockSpec((rows, D), lambda i: (i, 0))],
        out_specs=pl.BlockSpec((rows, D), lambda i: (i, 0)),
    )


@functools.lru_cache(maxsize=None)
def _make_gather(N, V, D):
    info = plsc.get_sparse_core_info()
    NC, NS = info.num_cores, info.num_subcores
    NW = NC * NS
    assert N % NW == 0
    n_per_w = N // NW
    CHUNK = 128
    assert n_per_w % CHUNK == 0
    n_chunks = n_per_w // CHUNK
    NBUF = 4
    assert n_chunks % NBUF == 0
    n_groups = n_chunks // NBUF

    mesh = plsc.VectorSubcoreMesh(core_axis_name="c", subcore_axis_name="s")

    @functools.partial(
        pl.kernel,
        mesh=mesh,
        out_type=jax.ShapeDtypeStruct((N, D), jnp.float32),
        scratch_types=[
            pltpu.VMEM((n_per_w,), jnp.int32),
            pltpu.VMEM((NBUF, CHUNK, D), jnp.float32),
        ]
        + [pltpu.SemaphoreType.DMA] * (2 * NBUF),
    )
    def k(x_hbm, tab_hbm, out_hbm, idx_v, bufs, *sems):
        gsems, ssems = sems[:NBUF], sems[NBUF:]
        wid = lax.axis_index("s") * NC + lax.axis_index("c")
        base = wid * n_per_w
        pltpu.sync_copy(x_hbm.at[pl.ds(base, n_per_w)], idx_v)

        def g_copy(i, b):
            return pltpu.make_async_copy(
                tab_hbm.at[idx_v.at[pl.ds(i * CHUNK, CHUNK)]],
                bufs.at[b],
                gsems[b],
            )

        def s_copy(i, b):
            return pltpu.make_async_copy(
                bufs.at[b],
                out_hbm.at[pl.ds(base + i * CHUNK, CHUNK)],
                ssems[b],
            )

        for b in range(NBUF):
            g_copy(b, b).start()

        def body(g, carry):
            for b in range(NBUF):
                i = g * NBUF + b
                g_copy(i, b).wait()
                s_copy(i, b).start()
            for b in range(NBUF):
                i = g * NBUF + b
                j = i + NBUF

                @pl.when(j < n_chunks)
                def _():
                    s_copy(i, b).wait()
                    g_copy(j, b).start()

            return carry

        lax.fori_loop(0, n_groups, body, 0)

        for b in range(NBUF):
            i = (n_groups - 1) * NBUF + b
            s_copy(i, b).wait()

    return k


def kernel(x, lut):
    B, H = x.shape
    V, D = lut.shape
    N = B * H
    scaled = _make_scale(V, D)(lut)
    xf = x.reshape(N).astype(jnp.int32)
    out = _make_gather(N, V, D)(xf, scaled)
    return out.reshape(B, H, D)


# speedup vs baseline: 1.1014x; 1.1014x over previous
import functools
import math

import jax
import jax.numpy as jnp
from jax import lax
from jax.experimental import pallas as pl
from jax.experimental.pallas import tpu as pltpu
from jax.experimental.pallas import tpu_sc as plsc

MODEL_DIM = 128
SCALE = math.sqrt(float(MODEL_DIM))


def _scale_body(lut_ref, out_ref):
    out_ref[...] = lut_ref[...] * SCALE


@functools.lru_cache(maxsize=None)
def _make_scale(V, D):
    rows = 5000
    assert V % rows == 0
    return pl.pallas_call(
        _scale_body,
        out_shape=jax.ShapeDtypeStruct((V, D), jnp.float32),
        grid=(V // rows,),
        in_specs=[pl.BlockSpec((rows, D), lambda i: (i, 0))],
        out_specs=pl.BlockSpec((rows, D), lambda i: (i, 0)),
    )


@functools.lru_cache(maxsize=None)
def _make_gather(N, V, D):
    info = plsc.get_sparse_core_info()
    NC, NS = info.num_cores, info.num_subcores
    NW = NC * NS
    assert N % NW == 0
    n_per_w = N // NW
    CHUNK = 128
    assert n_per_w % CHUNK == 0
    n_chunks = n_per_w // CHUNK
    NBUF = 5
    assert n_chunks % NBUF == 0
    n_groups = n_chunks // NBUF

    mesh = plsc.VectorSubcoreMesh(core_axis_name="c", subcore_axis_name="s")

    @functools.partial(
        pl.kernel,
        mesh=mesh,
        out_type=jax.ShapeDtypeStruct((N, D), jnp.float32),
        scratch_types=[
            pltpu.VMEM((n_per_w,), jnp.int32),
            pltpu.VMEM((NBUF, CHUNK, D), jnp.float32),
        ]
        + [pltpu.SemaphoreType.DMA] * (2 * NBUF),
    )
    def k(x_hbm, tab_hbm, out_hbm, idx_v, bufs, *sems):
        gsems, ssems = sems[:NBUF], sems[NBUF:]
        wid = lax.axis_index("s") * NC + lax.axis_index("c")
        base = wid * n_per_w
        pltpu.sync_copy(x_hbm.at[pl.ds(base, n_per_w)], idx_v)

        def g_copy(i, b):
            return pltpu.make_async_copy(
                tab_hbm.at[idx_v.at[pl.ds(i * CHUNK, CHUNK)]],
                bufs.at[b],
                gsems[b],
            )

        def s_copy(i, b):
            return pltpu.make_async_copy(
                bufs.at[b],
                out_hbm.at[pl.ds(base + i * CHUNK, CHUNK)],
                ssems[b],
            )

        for b in range(NBUF):
            g_copy(b, b).start()

        def body(g, carry):
            for b in range(NBUF):
                i = g * NBUF + b
                g_copy(i, b).wait()
                s_copy(i, b).start()
            for b in range(NBUF):
                i = g * NBUF + b
                j = i + NBUF

                @pl.when(j < n_chunks)
                def _():
                    s_copy(i, b).wait()
                    g_copy(j, b).start()

            return carry

        lax.fori_loop(0, n_groups, body, 0)

        for b in range(NBUF):
            i = (n_groups - 1) * NBUF + b
            s_copy(i, b).wait()

    return k


def kernel(x, lut):
    B, H = x.shape
    V, D = lut.shape
    N = B * H
    scaled = _make_scale(V, D)(lut)
    xf = x.reshape(N).astype(jnp.int32)
    out = _make_gather(N, V, D)(xf, scaled)
    return out.reshape(B, H, D)


# speedup vs baseline: 1.1111x; 1.0089x over previous
import functools
import math

import jax
import jax.numpy as jnp
from jax import lax
from jax.experimental import pallas as pl
from jax.experimental.pallas import tpu as pltpu
from jax.experimental.pallas import tpu_sc as plsc

MODEL_DIM = 128
SCALE = math.sqrt(float(MODEL_DIM))


def _scale_body(lut_ref, out_ref):
    out_ref[...] = lut_ref[...] * SCALE


@functools.lru_cache(maxsize=None)
def _make_scale(V, D):
    rows = 10000
    assert V % rows == 0
    return pl.pallas_call(
        _scale_body,
        out_shape=jax.ShapeDtypeStruct((V, D), jnp.float32),
        grid=(V // rows,),
        in_specs=[pl.BlockSpec((rows, D), lambda i: (i, 0))],
        out_specs=pl.BlockSpec((rows, D), lambda i: (i, 0)),
    )


@functools.lru_cache(maxsize=None)
def _make_gather(N, V, D):
    info = plsc.get_sparse_core_info()
    NC, NS = info.num_cores, info.num_subcores
    NW = NC * NS
    assert N % NW == 0
    n_per_w = N // NW
    CHUNK = 128
    assert n_per_w % CHUNK == 0
    n_chunks = n_per_w // CHUNK
    NBUF = 5
    assert n_chunks % NBUF == 0
    n_groups = n_chunks // NBUF

    mesh = plsc.VectorSubcoreMesh(core_axis_name="c", subcore_axis_name="s")

    @functools.partial(
        pl.kernel,
        mesh=mesh,
        out_type=jax.ShapeDtypeStruct((N, D), jnp.float32),
        scratch_types=[
            pltpu.VMEM((n_per_w,), jnp.int32),
            pltpu.VMEM((NBUF, CHUNK, D), jnp.float32),
        ]
        + [pltpu.SemaphoreType.DMA] * (2 * NBUF),
    )
    def k(x_hbm, tab_hbm, out_hbm, idx_v, bufs, *sems):
        gsems, ssems = sems[:NBUF], sems[NBUF:]
        wid = lax.axis_index("s") * NC + lax.axis_index("c")
        base = wid * n_per_w
        pltpu.sync_copy(x_hbm.at[pl.ds(base, n_per_w)], idx_v)

        def g_copy(i, b):
            return pltpu.make_async_copy(
                tab_hbm.at[idx_v.at[pl.ds(i * CHUNK, CHUNK)]],
                bufs.at[b],
                gsems[b],
            )

        def s_copy(i, b):
            return pltpu.make_async_copy(
                bufs.at[b],
                out_hbm.at[pl.ds(base + i * CHUNK, CHUNK)],
                ssems[b],
            )

        for b in range(NBUF):
            g_copy(b, b).start()

        def body(g, carry):
            for b in range(NBUF):
                i = g * NBUF + b
                g_copy(i, b).wait()
                s_copy(i, b).start()
            for b in range(NBUF):
                i = g * NBUF + b
                j = i + NBUF

                @pl.when(j < n_chunks)
                def _():
                    s_copy(i, b).wait()
                    g_copy(j, b).start()

            return carry

        lax.fori_loop(0, n_groups, body, 0)

        for b in range(NBUF):
            i = (n_groups - 1) * NBUF + b
            s_copy(i, b).wait()

    return k


def kernel(x, lut):
    B, H = x.shape
    V, D = lut.shape
    N = B * H
    scaled = _make_scale(V, D)(lut)
    xf = x.reshape(N).astype(jnp.int32)
    out = _make_gather(N, V, D)(xf, scaled)
    return out.reshape(B, H, D)


# speedup vs baseline: 1.1194x; 1.0075x over previous
import functools
import math

import jax
import jax.numpy as jnp
from jax import lax
from jax.experimental import pallas as pl
from jax.experimental.pallas import tpu as pltpu
from jax.experimental.pallas import tpu_sc as plsc

MODEL_DIM = 128
SCALE = math.sqrt(float(MODEL_DIM))


def _scale_body(lut_ref, out_ref):
    out_ref[...] = lut_ref[...] * SCALE


@functools.lru_cache(maxsize=None)
def _make_scale(V, D):
    rows = 10000
    assert V % rows == 0
    return pl.pallas_call(
        _scale_body,
        out_shape=jax.ShapeDtypeStruct((V, D), jnp.float32),
        grid=(V // rows,),
        in_specs=[pl.BlockSpec((rows, D), lambda i: (i, 0))],
        out_specs=pl.BlockSpec((rows, D), lambda i: (i, 0)),
    )


@functools.lru_cache(maxsize=None)
def _make_gather(N, V, D):
    info = plsc.get_sparse_core_info()
    NC, NS = info.num_cores, info.num_subcores
    NW = NC * NS
    assert N % NW == 0
    n_per_w = N // NW
    CHUNK = 128
    assert n_per_w % CHUNK == 0
    n_chunks = n_per_w // CHUNK
    NBUF = 5
    LOOKAHEAD = 3
    assert n_chunks % NBUF == 0
    n_groups = n_chunks // NBUF

    mesh = plsc.VectorSubcoreMesh(core_axis_name="c", subcore_axis_name="s")

    @functools.partial(
        pl.kernel,
        mesh=mesh,
        out_type=jax.ShapeDtypeStruct((N, D), jnp.float32),
        scratch_types=[
            pltpu.VMEM((n_per_w,), jnp.int32),
            pltpu.VMEM((NBUF, CHUNK, D), jnp.float32),
        ]
        + [pltpu.SemaphoreType.DMA] * (2 * NBUF),
    )
    def k(x_hbm, tab_hbm, out_hbm, idx_v, bufs, *sems):
        gsems, ssems = sems[:NBUF], sems[NBUF:]
        wid = lax.axis_index("s") * NC + lax.axis_index("c")
        base = wid * n_per_w
        pltpu.sync_copy(x_hbm.at[pl.ds(base, n_per_w)], idx_v)

        def g_copy(i, b):
            return pltpu.make_async_copy(
                tab_hbm.at[idx_v.at[pl.ds(i * CHUNK, CHUNK)]],
                bufs.at[b],
                gsems[b],
            )

        def s_copy(i, b):
            return pltpu.make_async_copy(
                bufs.at[b],
                out_hbm.at[pl.ds(base + i * CHUNK, CHUNK)],
                ssems[b],
            )

        for b in range(LOOKAHEAD):
            g_copy(b, b).start()

        def body(g, carry):
            for b in range(NBUF):
                i = g * NBUF + b
                g_copy(i, b).wait()
                s_copy(i, b).start()
                j = i + LOOKAHEAD
                bj = (b + LOOKAHEAD) % NBUF

                @pl.when(j < n_chunks)
                def _():
                    jp = j - NBUF

                    @pl.when(jp >= 0)
                    def _():
                        s_copy(lax.max(jp, 0), bj).wait()

                    g_copy(j, bj).start()

            return carry

        lax.fori_loop(0, n_groups, body, 0)

        for b in range(NBUF):
            i = n_chunks - NBUF + b
            s_copy(i, b).wait()

    return k


def kernel(x, lut):
    B, H = x.shape
    V, D = lut.shape
    N = B * H
    scaled = _make_scale(V, D)(lut)
    xf = x.reshape(N).astype(jnp.int32)
    out = _make_gather(N, V, D)(xf, scaled)
    return out.reshape(B, H, D)


# speedup vs baseline: 1.1211x; 1.0015x over previous
import functools
import math

import jax
import jax.numpy as jnp
from jax import lax
from jax.experimental import pallas as pl
from jax.experimental.pallas import tpu as pltpu
from jax.experimental.pallas import tpu_sc as plsc

MODEL_DIM = 128
SCALE = math.sqrt(float(MODEL_DIM))


def _scale_body(lut_ref, out_ref):
    out_ref[...] = lut_ref[...] * SCALE


@functools.lru_cache(maxsize=None)
def _make_scale(V, D):
    rows = 20000
    assert V % rows == 0
    return pl.pallas_call(
        _scale_body,
        out_shape=jax.ShapeDtypeStruct((V, D), jnp.float32),
        grid=(V // rows,),
        in_specs=[pl.BlockSpec((rows, D), lambda i: (i, 0))],
        out_specs=pl.BlockSpec((rows, D), lambda i: (i, 0)),
    )


@functools.lru_cache(maxsize=None)
def _make_gather(N, V, D):
    info = plsc.get_sparse_core_info()
    NC, NS = info.num_cores, info.num_subcores
    NW = NC * NS
    assert N % NW == 0
    n_per_w = N // NW
    CHUNK = 128
    assert n_per_w % CHUNK == 0
    n_chunks = n_per_w // CHUNK
    NBUF = 5
    LOOKAHEAD = 3
    assert n_chunks % NBUF == 0
    n_groups = n_chunks // NBUF

    mesh = plsc.VectorSubcoreMesh(core_axis_name="c", subcore_axis_name="s")

    @functools.partial(
        pl.kernel,
        mesh=mesh,
        out_type=jax.ShapeDtypeStruct((N, D), jnp.float32),
        scratch_types=[
            pltpu.VMEM((n_per_w,), jnp.int32),
            pltpu.VMEM((NBUF, CHUNK, D), jnp.float32),
        ]
        + [pltpu.SemaphoreType.DMA] * (2 * NBUF),
    )
    def k(x_hbm, tab_hbm, out_hbm, idx_v, bufs, *sems):
        gsems, ssems = sems[:NBUF], sems[NBUF:]
        wid = lax.axis_index("s") * NC + lax.axis_index("c")
        base = wid * n_per_w
        pltpu.sync_copy(x_hbm.at[pl.ds(base, n_per_w)], idx_v)

        def g_copy(i, b):
            return pltpu.make_async_copy(
                tab_hbm.at[idx_v.at[pl.ds(i * CHUNK, CHUNK)]],
                bufs.at[b],
                gsems[b],
            )

        def s_copy(i, b):
            return pltpu.make_async_copy(
                bufs.at[b],
                out_hbm.at[pl.ds(base + i * CHUNK, CHUNK)],
                ssems[b],
            )

        for b in range(LOOKAHEAD):
            g_copy(b, b).start()

        def body(g, carry):
            for b in range(NBUF):
                i = g * NBUF + b
                g_copy(i, b).wait()
                s_copy(i, b).start()
                j = i + LOOKAHEAD
                bj = (b + LOOKAHEAD) % NBUF

                @pl.when(j < n_chunks)
                def _():
                    jp = j - NBUF

                    @pl.when(jp >= 0)
                    def _():
                        s_copy(lax.max(jp, 0), bj).wait()

                    g_copy(j, bj).start()

            return carry

        lax.fori_loop(0, n_groups, body, 0)

        for b in range(NBUF):
            i = n_chunks - NBUF + b
            s_copy(i, b).wait()

    return k


def kernel(x, lut):
    B, H = x.shape
    V, D = lut.shape
    N = B * H
    scaled = _make_scale(V, D)(lut)
    xf = x.reshape(N).astype(jnp.int32)
    out = _make_gather(N, V, D)(xf, scaled)
    return out.reshape(B, H, D)


# speedup vs baseline: 1.1243x; 1.0029x over previous
import functools
import math

import jax
import jax.numpy as jnp
from jax import lax
from jax.experimental import pallas as pl
from jax.experimental.pallas import tpu as pltpu
from jax.experimental.pallas import tpu_sc as plsc

MODEL_DIM = 128
SCALE = math.sqrt(float(MODEL_DIM))


def _scale_body(lut_ref, out_ref):
    out_ref[...] = lut_ref[...] * SCALE


@functools.lru_cache(maxsize=None)
def _make_scale(V, D):
    rows = 25000
    assert V % rows == 0
    return pl.pallas_call(
        _scale_body,
        out_shape=jax.ShapeDtypeStruct((V, D), jnp.float32),
        grid=(V // rows,),
        in_specs=[pl.BlockSpec((rows, D), lambda i: (i, 0))],
        out_specs=pl.BlockSpec((rows, D), lambda i: (i, 0)),
    )


@functools.lru_cache(maxsize=None)
def _make_gather(N, V, D):
    info = plsc.get_sparse_core_info()
    NC, NS = info.num_cores, info.num_subcores
    NW = NC * NS
    assert N % NW == 0
    n_per_w = N // NW
    CHUNK = 128
    assert n_per_w % CHUNK == 0
    n_chunks = n_per_w // CHUNK
    NBUF = 5
    LOOKAHEAD = 3
    assert n_chunks % NBUF == 0
    n_groups = n_chunks // NBUF

    mesh = plsc.VectorSubcoreMesh(core_axis_name="c", subcore_axis_name="s")

    @functools.partial(
        pl.kernel,
        mesh=mesh,
        out_type=jax.ShapeDtypeStruct((N, D), jnp.float32),
        scratch_types=[
            pltpu.VMEM((n_per_w,), jnp.int32),
            pltpu.VMEM((NBUF, CHUNK, D), jnp.float32),
        ]
        + [pltpu.SemaphoreType.DMA] * (2 * NBUF),
    )
    def k(x_hbm, tab_hbm, out_hbm, idx_v, bufs, *sems):
        gsems, ssems = sems[:NBUF], sems[NBUF:]
        wid = lax.axis_index("s") * NC + lax.axis_index("c")
        base = wid * n_per_w
        pltpu.sync_copy(x_hbm.at[pl.ds(base, n_per_w)], idx_v)

        def g_copy(i, b):
            return pltpu.make_async_copy(
                tab_hbm.at[idx_v.at[pl.ds(i * CHUNK, CHUNK)]],
                bufs.at[b],
                gsems[b],
            )

        def s_copy(i, b):
            return pltpu.make_async_copy(
                bufs.at[b],
                out_hbm.at[pl.ds(base + i * CHUNK, CHUNK)],
                ssems[b],
            )

        for b in range(LOOKAHEAD):
            g_copy(b, b).start()

        def body(g, carry):
            for b in range(NBUF):
                i = g * NBUF + b
                g_copy(i, b).wait()
                s_copy(i, b).start()
                j = i + LOOKAHEAD
                bj = (b + LOOKAHEAD) % NBUF

                @pl.when(j < n_chunks)
                def _():
                    jp = j - NBUF

                    @pl.when(jp >= 0)
                    def _():
                        s_copy(lax.max(jp, 0), bj).wait()

                    g_copy(j, bj).start()

            return carry

        lax.fori_loop(0, n_groups, body, 0)

        for b in range(NBUF):
            i = n_chunks - NBUF + b
            s_copy(i, b).wait()

    return k


def kernel(x, lut):
    B, H = x.shape
    V, D = lut.shape
    N = B * H
    scaled = _make_scale(V, D)(lut)
    xf = x.reshape(N).astype(jnp.int32)
    out = _make_gather(N, V, D)(xf, scaled)
    return out.reshape(B, H, D)
